# baseline (speedup 1.0000x reference)
import functools

import jax
import jax.numpy as jnp
from jax.experimental import pallas as pl
from jax.experimental.pallas import tpu as pltpu

_N = 4096
_W = 1024
_B = 2048


def _tc_body(po_ref, ri_ref, out_ref):
    po = po_ref[...]
    out_ref[0] = po + ri_ref[0:1]
    out_ref[1] = po + ri_ref[1:2]


@jax.jit
def _run(po_table, ri_table):
    out = pl.pallas_call(
        _tc_body,
        grid=(_N // _B,),
        in_specs=[
            pl.BlockSpec((_B, _W), lambda i: (i, 0)),
            pl.BlockSpec((2, _W), lambda i: (0, 0)),
        ],
        out_specs=pl.BlockSpec((2, _B, _W), lambda i: (0, i, 0)),
        out_shape=jax.ShapeDtypeStruct((2, _N, _W), jnp.float32),
    )(po_table, ri_table)
    return out


def kernel(po_table, ri_table, po_idx, ri_idx):
    out = _run(po_table, ri_table)
    return out.reshape(1, 2 * _N, _W)
